# Initial kernel scaffold; baseline (speedup 1.0000x reference)
import jax
import jax.numpy as jnp
from jax.experimental import pallas as pl


def kernel(classifications, regressions, anchors, annotations):
    raise NotImplementedError("write your pallas kernel here")


# speedup vs baseline: 1.1909x; 1.1909x over previous
import jax
import jax.numpy as jnp
from jax import lax
from jax.experimental import pallas as pl
from jax.experimental.pallas import tpu as pltpu

ALPHA = 0.25
GAMMA = 2.0
NB = 2000


def _body(c_ref, r_ref, a_ref, ann_ref, cls_out, reg_out, npos_out):
    n = pl.program_id(1)
    nb = c_ref.shape[1]
    a = a_ref[0]
    bb = ann_ref[0]
    p = jnp.clip(c_ref[0], 1e-4, 1.0 - 1e-4)
    reg = r_ref[0]
    M = bb.shape[1]
    C = p.shape[1]

    ax0 = a[:, 0:1]
    ay0 = a[:, 1:2]
    ax1 = a[:, 2:3]
    ay1 = a[:, 3:4]
    bx0 = bb[0:1, :]
    by0 = bb[1:2, :]
    bx1 = bb[2:3, :]
    by1 = bb[3:4, :]
    bcl = bb[4:5, :]

    area_b = (bx1 - bx0) * (by1 - by0)
    iw = jnp.minimum(ax1, bx1) - jnp.maximum(ax0, bx0)
    ih = jnp.minimum(ay1, by1) - jnp.maximum(ay0, by0)
    iw = jnp.maximum(iw, 0.0)
    ih = jnp.maximum(ih, 0.0)
    ua = (ax1 - ax0) * (ay1 - ay0) + area_b - iw * ih
    ua = jnp.maximum(ua, 1e-8)
    iou = iw * ih / ua
    iou = jnp.where(bcl != -1.0, iou, -1.0)

    iou_max = jnp.max(iou, axis=1, keepdims=True)
    am = jnp.argmax(iou, axis=1).astype(jnp.int32).reshape(nb, 1)
    onehot = lax.broadcasted_iota(jnp.int32, (nb, M), 1) == am

    def gather(row):
        return jnp.sum(jnp.where(onehot, row, 0.0), axis=1, keepdims=True)

    as0 = gather(bx0)
    as1 = gather(by0)
    as2 = gather(bx1)
    as3 = gather(by1)
    as4 = gather(bcl)

    pos = iou_max >= 0.5
    care = iou_max < 0.4
    cls_id = as4.astype(jnp.int32)

    l0 = (1.0 - ALPHA) * p * p * (-jnp.log(1.0 - p))
    row_l0 = jnp.sum(l0, axis=1, keepdims=True)

    csel = lax.broadcasted_iota(jnp.int32, (nb, C), 1) == cls_id
    p_sel = jnp.sum(jnp.where(csel, p, 0.0), axis=1, keepdims=True)
    p_sel = jnp.clip(p_sel, 1e-4, 1.0 - 1e-4)
    l1 = ALPHA * (1.0 - p_sel) * (1.0 - p_sel) * (-jnp.log(p_sel))
    l0_sel = (1.0 - ALPHA) * p_sel * p_sel * (-jnp.log(1.0 - p_sel))
    corr = l1 - l0_sel

    cls_part = (jnp.sum(jnp.where(care | pos, row_l0, 0.0))
                + jnp.sum(jnp.where(pos, corr, 0.0)))
    npos_part = jnp.sum(pos.astype(jnp.float32))

    aw = ax1 - ax0
    ah = ay1 - ay0
    acx = ax0 + 0.5 * aw
    acy = ay0 + 0.5 * ah
    gw_u = as2 - as0
    gh_u = as3 - as1
    gcx = as0 + 0.5 * gw_u
    gcy = as1 + 0.5 * gh_u
    gw = jnp.maximum(gw_u, 1.0)
    gh = jnp.maximum(gh_u, 1.0)
    tdx = (gcx - acx) / aw * 10.0
    tdy = (gcy - acy) / ah * 10.0
    tdw = jnp.log(gw / aw) * 5.0
    tdh = jnp.log(gh / ah) * 5.0
    tgt = jnp.concatenate([tdx, tdy, tdw, tdh], axis=1)
    diff = jnp.abs(tgt - reg)
    rl = jnp.where(diff <= 1.0 / 9.0, 0.5 * 9.0 * diff * diff, diff - 0.5 / 9.0)
    reg_part = jnp.sum(jnp.where(pos, rl, 0.0))

    @pl.when(n == 0)
    def _():
        cls_out[0, 0, 0] = 0.0
        reg_out[0, 0, 0] = 0.0
        npos_out[0, 0, 0] = 0.0

    cls_out[0, 0, 0] += cls_part
    reg_out[0, 0, 0] += reg_part
    npos_out[0, 0, 0] += npos_part


def kernel(classifications, regressions, anchors, annotations):
    B, N, C = classifications.shape
    M = annotations.shape[1]
    K = N // NB
    ann_t = jnp.transpose(annotations, (0, 2, 1))

    f32 = jnp.float32
    cls_sum, reg_sum, npos = pl.pallas_call(
        _body,
        grid=(B, K),
        in_specs=[
            pl.BlockSpec((1, NB, C), lambda j, n: (j, n, 0)),
            pl.BlockSpec((1, NB, 4), lambda j, n: (j, n, 0)),
            pl.BlockSpec((1, NB, 4), lambda j, n: (0, n, 0)),
            pl.BlockSpec((1, 5, M), lambda j, n: (j, 0, 0)),
        ],
        out_specs=[
            pl.BlockSpec((1, 1, 1), lambda j, n: (j, 0, 0), memory_space=pltpu.SMEM),
            pl.BlockSpec((1, 1, 1), lambda j, n: (j, 0, 0), memory_space=pltpu.SMEM),
            pl.BlockSpec((1, 1, 1), lambda j, n: (j, 0, 0), memory_space=pltpu.SMEM),
        ],
        out_shape=[
            jax.ShapeDtypeStruct((B, 1, 1), f32),
            jax.ShapeDtypeStruct((B, 1, 1), f32),
            jax.ShapeDtypeStruct((B, 1, 1), f32),
        ],
    )(classifications, regressions, anchors, ann_t)

    cls_sum = cls_sum[:, 0, 0]
    reg_sum = reg_sum[:, 0, 0]
    npos = npos[:, 0, 0]
    cls_losses = cls_sum / jnp.maximum(npos, 1.0)
    reg_losses = jnp.where(npos > 0.0, reg_sum / jnp.maximum(npos * 4.0, 1.0), 0.0)
    return (jnp.mean(cls_losses, keepdims=True), jnp.mean(reg_losses, keepdims=True))


# speedup vs baseline: 2.9792x; 2.5016x over previous
import jax
import jax.numpy as jnp
from jax import lax
from jax.experimental import pallas as pl
from jax.experimental.pallas import tpu as pltpu

ALPHA = 0.25
GAMMA = 2.0
NB = 4000


def _body(c_ref, rT_ref, aT_ref, annT_ref, annm_ref, cls_out, reg_out, npos_out):
    n = pl.program_id(1)
    nb = c_ref.shape[1]
    aT = jnp.transpose(aT_ref[0], (1, 0))
    annT = annT_ref[0]
    annm = annm_ref[0]
    rT = jnp.transpose(rT_ref[0], (1, 0))
    p = jnp.clip(c_ref[0], 1e-4, 1.0 - 1e-4)
    M = annm.shape[0]
    C = p.shape[1]
    f32 = jnp.float32

    ax0 = aT[0:1, :]
    ay0 = aT[1:2, :]
    ax1 = aT[2:3, :]
    ay1 = aT[3:4, :]
    bx0 = annm[:, 0:1]
    by0 = annm[:, 1:2]
    bx1 = annm[:, 2:3]
    by1 = annm[:, 3:4]
    bcl = annm[:, 4:5]

    area_b = (bx1 - bx0) * (by1 - by0)
    iw = jnp.minimum(ax1, bx1) - jnp.maximum(ax0, bx0)
    ih = jnp.minimum(ay1, by1) - jnp.maximum(ay0, by0)
    iw = jnp.maximum(iw, 0.0)
    ih = jnp.maximum(ih, 0.0)
    ua = (ax1 - ax0) * (ay1 - ay0) + area_b - iw * ih
    ua = jnp.maximum(ua, 1e-8)
    iou = iw * ih / ua
    iou = jnp.where(bcl != -1.0, iou, -1.0)

    iou_max = jnp.max(iou, axis=0, keepdims=True)
    am = jnp.argmax(iou, axis=0).astype(jnp.int32).reshape(1, nb)
    onehot = (lax.broadcasted_iota(jnp.int32, (M, nb), 0) == am).astype(f32)

    assigned = jnp.dot(annT, onehot, preferred_element_type=f32)

    pos = iou_max >= 0.5
    w_r = ((iou_max < 0.4) | pos).astype(f32)
    posf = pos.astype(f32)
    npos_part = jnp.sum(posf)

    BC = (lax.broadcasted_iota(jnp.int32, (M, C), 1) == bcl.astype(jnp.int32)).astype(f32)
    csel = lax.dot_general(onehot, BC, (((0,), (0,)), ((), ())),
                           preferred_element_type=f32)

    l0 = (1.0 - ALPHA) * p * p * (-jnp.log(1.0 - p))
    ones_c = jnp.ones((C, 1), dtype=f32)
    row_l0 = jnp.dot(l0, ones_c, preferred_element_type=f32)
    p_sel = jnp.dot(p * csel, ones_c, preferred_element_type=f32)
    l0_sel = jnp.dot(l0 * csel, ones_c, preferred_element_type=f32)

    p_sel = jnp.maximum(p_sel, 1e-4)
    om = 1.0 - p_sel
    l1 = (ALPHA * om * om) * (-jnp.log(p_sel))
    corr = l1 - l0_sel

    cls_part = (jnp.dot(w_r, row_l0, preferred_element_type=f32)[0, 0]
                + jnp.dot(posf, corr, preferred_element_type=f32)[0, 0])

    aw = ax1 - ax0
    ah = ay1 - ay0
    acx = ax0 + 0.5 * aw
    acy = ay0 + 0.5 * ah
    gx0 = assigned[0:1, :]
    gy0 = assigned[1:2, :]
    gx1 = assigned[2:3, :]
    gy1 = assigned[3:4, :]
    gw_u = gx1 - gx0
    gh_u = gy1 - gy0
    gcx = gx0 + 0.5 * gw_u
    gcy = gy0 + 0.5 * gh_u
    gw = jnp.maximum(gw_u, 1.0)
    gh = jnp.maximum(gh_u, 1.0)
    tdx = (gcx - acx) / aw * 10.0
    tdy = (gcy - acy) / ah * 10.0
    tdw = jnp.log(gw / aw) * 5.0
    tdh = jnp.log(gh / ah) * 5.0

    def sl1(tk, k):
        d = jnp.abs(tk - rT[k:k + 1, :])
        return jnp.where(d <= 1.0 / 9.0, 4.5 * d * d, d - 0.5 / 9.0)

    rl = sl1(tdx, 0) + sl1(tdy, 1) + sl1(tdw, 2) + sl1(tdh, 3)
    reg_part = jnp.sum(jnp.where(pos, rl, 0.0))

    @pl.when(n == 0)
    def _():
        cls_out[0, 0, 0] = 0.0
        reg_out[0, 0, 0] = 0.0
        npos_out[0, 0, 0] = 0.0

    cls_out[0, 0, 0] += cls_part
    reg_out[0, 0, 0] += reg_part
    npos_out[0, 0, 0] += npos_part


def kernel(classifications, regressions, anchors, annotations):
    B, N, C = classifications.shape
    M = annotations.shape[1]
    K = N // NB
    ann_t = jnp.transpose(annotations, (0, 2, 1))

    f32 = jnp.float32
    cls_sum, reg_sum, npos = pl.pallas_call(
        _body,
        grid=(B, K),
        in_specs=[
            pl.BlockSpec((1, NB, C), lambda j, n: (j, n, 0)),
            pl.BlockSpec((1, NB, 4), lambda j, n: (j, n, 0)),
            pl.BlockSpec((1, NB, 4), lambda j, n: (0, n, 0)),
            pl.BlockSpec((1, 5, M), lambda j, n: (j, 0, 0)),
            pl.BlockSpec((1, M, 5), lambda j, n: (j, 0, 0)),
        ],
        out_specs=[
            pl.BlockSpec((1, 1, 1), lambda j, n: (j, 0, 0), memory_space=pltpu.SMEM),
            pl.BlockSpec((1, 1, 1), lambda j, n: (j, 0, 0), memory_space=pltpu.SMEM),
            pl.BlockSpec((1, 1, 1), lambda j, n: (j, 0, 0), memory_space=pltpu.SMEM),
        ],
        out_shape=[
            jax.ShapeDtypeStruct((B, 1, 1), f32),
            jax.ShapeDtypeStruct((B, 1, 1), f32),
            jax.ShapeDtypeStruct((B, 1, 1), f32),
        ],
    )(classifications, regressions, anchors, ann_t, annotations)

    cls_sum = cls_sum[:, 0, 0]
    reg_sum = reg_sum[:, 0, 0]
    npos = npos[:, 0, 0]
    cls_losses = cls_sum / jnp.maximum(npos, 1.0)
    reg_losses = jnp.where(npos > 0.0, reg_sum / jnp.maximum(npos * 4.0, 1.0), 0.0)
    return (jnp.mean(cls_losses, keepdims=True), jnp.mean(reg_losses, keepdims=True))
